# baseline (speedup 1.0000x reference)
import functools

import jax
import jax.numpy as jnp
from jax import lax
from jax.experimental import pallas as pl
from jax.experimental.pallas import tpu as pltpu
from jax.experimental.pallas import tpu_sc as plsc

_B, _DIN, _D, _C = 16384, 128, 64, 8

_NC, _NS = 2, 16
_NW = _NC * _NS
_NPAIR = _B // 2
_PPW = _NPAIR // _NW
_CHUNK = 128
_NCHUNK = _PPW // _CHUNK
_REP = 32


def _sc_mask_gather_body(table_hbm, idx_hbm, out_hbm, idx_v, rows_v, sem, sem_out):
    wid = lax.axis_index("s") * _NC + lax.axis_index("c")
    pltpu.sync_copy(idx_hbm.at[pl.ds(wid * _NCHUNK, _NCHUNK)], idx_v)
    cps = [
        pltpu.async_copy(table_hbm.at[idx_v.at[j]], rows_v.at[j], sem)
        for j in range(_NCHUNK)
    ]
    outs = []
    for j in range(_NCHUNK):
        cps[j].wait()
        outs.append(
            pltpu.async_copy(
                rows_v.at[j],
                out_hbm.at[pl.ds(wid * _PPW + j * _CHUNK, _CHUNK)],
                sem_out,
            )
        )
    for cp in outs:
        cp.wait()


@functools.cache
def _sc_mask_gather():
    return pl.kernel(
        _sc_mask_gather_body,
        mesh=plsc.VectorSubcoreMesh(core_axis_name="c", subcore_axis_name="s"),
        out_type=jax.ShapeDtypeStruct((_NPAIR, 2 * _D), jnp.float32),
        scratch_types=[
            pltpu.VMEM((_NCHUNK, _CHUNK), jnp.int32),
            pltpu.VMEM((_NCHUNK, _CHUNK, 2 * _D), jnp.float32),
            pltpu.SemaphoreType.DMA,
            pltpu.SemaphoreType.DMA,
        ],
    )


_BM = 2048


def _tc_body(x_ref, m_ref, w_ref, b_ref,
             emb_ref, masked_ref, masknorm_ref, embnorm_ref,
             acc_mask, acc_sq):
    i = pl.program_id(0)

    @pl.when(i == 0)
    def _init():
        acc_mask[0] = 0.0
        acc_sq[0] = 0.0

    y = jnp.dot(x_ref[...], w_ref[...], preferred_element_type=jnp.float32)
    y = y + b_ref[...]
    m = jnp.maximum(m_ref[...], 0.0)
    t = y * m
    s = jnp.sum(t * t, axis=1, keepdims=True)
    inv = 1.0 / (jnp.sqrt(s) + 1e-10)
    emb_ref[...] = y
    masked_ref[...] = t * inv

    acc_mask[0] += jnp.sum(m)
    acc_sq[0] += jnp.sum(y * y)

    @pl.when(i == pl.num_programs(0) - 1)
    def _fin():
        masknorm_ref[0, 0] = acc_mask[0]
        embnorm_ref[0, 0] = jnp.sqrt(acc_sq[0])


def _tc_call(x, mask_rows, w, b2, interpret=False):
    grid = _B // _BM
    return pl.pallas_call(
        _tc_body,
        grid=(grid,),
        in_specs=[
            pl.BlockSpec((_BM, _DIN), lambda i: (i, 0)),
            pl.BlockSpec((_BM, _D), lambda i: (i, 0)),
            pl.BlockSpec((_DIN, _D), lambda i: (0, 0)),
            pl.BlockSpec((1, _D), lambda i: (0, 0)),
        ],
        out_specs=[
            pl.BlockSpec((_BM, _D), lambda i: (i, 0)),
            pl.BlockSpec((_BM, _D), lambda i: (i, 0)),
            pl.BlockSpec(memory_space=pltpu.SMEM),
            pl.BlockSpec(memory_space=pltpu.SMEM),
        ],
        out_shape=[
            jax.ShapeDtypeStruct((_B, _D), jnp.float32),
            jax.ShapeDtypeStruct((_B, _D), jnp.float32),
            jax.ShapeDtypeStruct((1, 1), jnp.float32),
            jax.ShapeDtypeStruct((1, 1), jnp.float32),
        ],
        scratch_shapes=[
            pltpu.SMEM((1,), jnp.float32),
            pltpu.SMEM((1,), jnp.float32),
        ],
        interpret=interpret,
    )(x, mask_rows, w, b2)


def kernel(x, c, W, b, masks_weight):
    ptab = jnp.concatenate(
        [
            jnp.broadcast_to(masks_weight[:, None, :], (_C, _C, _D)),
            jnp.broadcast_to(masks_weight[None, :, :], (_C, _C, _D)),
        ],
        axis=-1,
    ).reshape(_C * _C, 2 * _D)
    ptab_rep = jnp.tile(ptab, (_REP, 1))
    c2 = c.astype(jnp.int32).reshape(_NPAIR, 2)
    pair_idx = (
        c2[:, 0] * _C + c2[:, 1]
        + _C * _C * (jnp.arange(_NPAIR, dtype=jnp.int32) % _REP)
    ).reshape(_NPAIR // _CHUNK, _CHUNK)
    pair_rows = _sc_mask_gather()(ptab_rep, pair_idx)
    mask_rows = pair_rows.reshape(_B, _D)
    emb, masked, masknorm, embnorm = _tc_call(x, mask_rows, W, b.reshape(1, _D))
    return masked, masknorm.reshape(()), embnorm.reshape(()), emb


# speedup vs baseline: 1.5060x; 1.5060x over previous
import functools

import jax
import jax.numpy as jnp
from jax import lax
from jax.experimental import pallas as pl
from jax.experimental.pallas import tpu as pltpu
from jax.experimental.pallas import tpu_sc as plsc

_B, _DIN, _D, _C = 16384, 128, 64, 8

_NC, _NS = 2, 16
_NW = _NC * _NS
_BPW = _B // _NW
_CHUNK = 128
_NCHUNK = _BPW // _CHUNK


_REP = 256


def _sc_mask_gather_body(table_hbm, idx_hbm, out_hbm, idx_v, rows_v, sem, sem_out):
    wid = lax.axis_index("s") * _NC + lax.axis_index("c")
    base = wid * _BPW
    pltpu.sync_copy(idx_hbm.at[pl.ds(wid * _NCHUNK, _NCHUNK)], idx_v)
    cps = [
        pltpu.async_copy(table_hbm.at[idx_v.at[j]], rows_v.at[j], sem)
        for j in range(_NCHUNK)
    ]
    outs = []
    for j in range(_NCHUNK):
        cps[j].wait()
        outs.append(
            pltpu.async_copy(
                rows_v.at[j], out_hbm.at[pl.ds(base + j * _CHUNK, _CHUNK)], sem_out
            )
        )
    for cp in outs:
        cp.wait()


@functools.cache
def _sc_mask_gather():
    return pl.kernel(
        _sc_mask_gather_body,
        mesh=plsc.VectorSubcoreMesh(core_axis_name="c", subcore_axis_name="s"),
        out_type=jax.ShapeDtypeStruct((_B, 128), jnp.float32),
        scratch_types=[
            pltpu.VMEM((_NCHUNK, _CHUNK), jnp.int32),
            pltpu.VMEM((_NCHUNK, _CHUNK, 128), jnp.float32),
            pltpu.SemaphoreType.DMA,
            pltpu.SemaphoreType.DMA,
        ],
    )


_BM = 2048


def _tc_body(x_ref, m_ref, w_ref, b_ref,
             emb_ref, masked_ref, masknorm_ref, embnorm_ref,
             acc_mask, acc_sq):
    i = pl.program_id(0)

    @pl.when(i == 0)
    def _init():
        acc_mask[0] = 0.0
        acc_sq[0] = 0.0

    y = lax.dot_general(
        x_ref[...], w_ref[...], (((1,), (1,)), ((), ())),
        preferred_element_type=jnp.float32,
    )
    y = y + b_ref[...]
    m = jnp.maximum(m_ref[:, : _D], 0.0)
    t = y * m
    s = jnp.sum(t * t, axis=1, keepdims=True)
    inv = 1.0 / (jnp.sqrt(s) + 1e-10)
    emb_ref[...] = y.T
    masked_ref[...] = (t * inv).T

    acc_mask[0] += jnp.sum(m)
    acc_sq[0] += jnp.sum(y * y)

    @pl.when(i == pl.num_programs(0) - 1)
    def _fin():
        masknorm_ref[0, 0] = acc_mask[0]
        embnorm_ref[0, 0] = jnp.sqrt(acc_sq[0])


def _tc_call(x, mask_rows, w, b2, interpret=False):
    grid = _B // _BM
    return pl.pallas_call(
        _tc_body,
        grid=(grid,),
        in_specs=[
            pl.BlockSpec((_BM, _DIN), lambda i: (i, 0)),
            pl.BlockSpec((_BM, 128), lambda i: (i, 0)),
            pl.BlockSpec((_D, _DIN), lambda i: (0, 0)),
            pl.BlockSpec((1, _D), lambda i: (0, 0)),
        ],
        out_specs=[
            pl.BlockSpec((_D, _BM), lambda i: (0, i)),
            pl.BlockSpec((_D, _BM), lambda i: (0, i)),
            pl.BlockSpec(memory_space=pltpu.SMEM),
            pl.BlockSpec(memory_space=pltpu.SMEM),
        ],
        out_shape=[
            jax.ShapeDtypeStruct((_D, _B), jnp.float32),
            jax.ShapeDtypeStruct((_D, _B), jnp.float32),
            jax.ShapeDtypeStruct((1, 1), jnp.float32),
            jax.ShapeDtypeStruct((1, 1), jnp.float32),
        ],
        scratch_shapes=[
            pltpu.SMEM((1,), jnp.float32),
            pltpu.SMEM((1,), jnp.float32),
        ],
        interpret=interpret,
    )(x, mask_rows, w, b2)


def kernel(x, c, W, b, masks_weight):
    table_pad = jnp.pad(masks_weight, ((0, 0), (0, 128 - _D)))
    table_rep = jnp.tile(table_pad, (_REP, 1))
    cc = c.astype(jnp.int32)
    c_adj = (cc + _C * (jnp.arange(_B, dtype=jnp.int32) % _REP)).reshape(
        _B // _CHUNK, _CHUNK)
    mask_rows = _sc_mask_gather()(table_rep, c_adj)
    emb_t, masked_t, masknorm, embnorm = _tc_call(
        x, mask_rows, W.T, b.reshape(1, _D))
    return (masked_t.T, masknorm.reshape(()), embnorm.reshape(()), emb_t.T)


# speedup vs baseline: 1.5489x; 1.0285x over previous
import functools

import jax
import jax.numpy as jnp
from jax import lax
from jax.experimental import pallas as pl
from jax.experimental.pallas import tpu as pltpu
from jax.experimental.pallas import tpu_sc as plsc

_B, _DIN, _D, _C = 16384, 128, 64, 8

_NC, _NS = 2, 16
_NW = _NC * _NS
_BPW = _B // _NW
_CHUNK = 128
_NCHUNK = _BPW // _CHUNK
_REP = 256


def _sc_mask_gather_body(table_hbm, idx_hbm, out_hbm, idx_v, rows_v, sem, sem_out):
    wid = lax.axis_index("s") * _NC + lax.axis_index("c")
    base = wid * _BPW
    pltpu.sync_copy(idx_hbm.at[pl.ds(wid * _NCHUNK, _NCHUNK)], idx_v)
    cps = [
        pltpu.async_copy(table_hbm.at[idx_v.at[j]], rows_v.at[j], sem)
        for j in range(_NCHUNK)
    ]
    outs = []
    for j in range(_NCHUNK):
        cps[j].wait()
        outs.append(
            pltpu.async_copy(
                rows_v.at[j], out_hbm.at[pl.ds(base + j * _CHUNK, _CHUNK)], sem_out
            )
        )
    for cp in outs:
        cp.wait()


@functools.cache
def _sc_mask_gather():
    return pl.kernel(
        _sc_mask_gather_body,
        mesh=plsc.VectorSubcoreMesh(core_axis_name="c", subcore_axis_name="s"),
        out_type=jax.ShapeDtypeStruct((_B, 128), jnp.float32),
        scratch_types=[
            pltpu.VMEM((_NCHUNK, _CHUNK), jnp.int32),
            pltpu.VMEM((_NCHUNK, _CHUNK, 128), jnp.float32),
            pltpu.SemaphoreType.DMA,
            pltpu.SemaphoreType.DMA,
        ],
    )


_BM = 2048


def _tc_matmul_body(x_ref, w_ref, b_ref, emb_ref, embnorm_ref, acc_sq):
    i = pl.program_id(0)

    @pl.when(i == 0)
    def _init():
        acc_sq[0] = 0.0

    y = lax.dot_general(
        x_ref[...], w_ref[...], (((1,), (1,)), ((), ())),
        preferred_element_type=jnp.float32,
    )
    y = y + b_ref[...]
    emb_ref[...] = y.T
    acc_sq[0] += jnp.sum(y * y)

    @pl.when(i == pl.num_programs(0) - 1)
    def _fin():
        embnorm_ref[0, 0] = jnp.sqrt(acc_sq[0])


def _tc_matmul(x, wt, b2, interpret=False):
    return pl.pallas_call(
        _tc_matmul_body,
        grid=(_B // _BM,),
        in_specs=[
            pl.BlockSpec((_BM, _DIN), lambda i: (i, 0)),
            pl.BlockSpec((_D, _DIN), lambda i: (0, 0)),
            pl.BlockSpec((1, _D), lambda i: (0, 0)),
        ],
        out_specs=[
            pl.BlockSpec((_D, _BM), lambda i: (0, i)),
            pl.BlockSpec(memory_space=pltpu.SMEM),
        ],
        out_shape=[
            jax.ShapeDtypeStruct((_D, _B), jnp.float32),
            jax.ShapeDtypeStruct((1, 1), jnp.float32),
        ],
        scratch_shapes=[pltpu.SMEM((1,), jnp.float32)],
        interpret=interpret,
    )(x, wt, b2)


def _tc_epilogue_body(embt_ref, m_ref, masked_ref, masknorm_ref, acc_mask):
    i = pl.program_id(0)

    @pl.when(i == 0)
    def _init():
        acc_mask[0] = 0.0

    y_t = embt_ref[...]
    m = jnp.maximum(m_ref[:, : _D], 0.0)
    t = y_t * m.T
    s = jnp.sum(t * t, axis=0, keepdims=True)
    inv = 1.0 / (jnp.sqrt(s) + 1e-10)
    masked_ref[...] = t * inv
    acc_mask[0] += jnp.sum(m)

    @pl.when(i == pl.num_programs(0) - 1)
    def _fin():
        masknorm_ref[0, 0] = acc_mask[0]


def _tc_epilogue(emb_t, mask_rows, interpret=False):
    return pl.pallas_call(
        _tc_epilogue_body,
        grid=(_B // _BM,),
        in_specs=[
            pl.BlockSpec((_D, _BM), lambda i: (0, i)),
            pl.BlockSpec((_BM, 128), lambda i: (i, 0)),
        ],
        out_specs=[
            pl.BlockSpec((_D, _BM), lambda i: (0, i)),
            pl.BlockSpec(memory_space=pltpu.SMEM),
        ],
        out_shape=[
            jax.ShapeDtypeStruct((_D, _B), jnp.float32),
            jax.ShapeDtypeStruct((1, 1), jnp.float32),
        ],
        scratch_shapes=[pltpu.SMEM((1,), jnp.float32)],
        interpret=interpret,
    )(emb_t, mask_rows)


def kernel(x, c, W, b, masks_weight):
    table_pad = jnp.pad(masks_weight, ((0, 0), (0, 128 - _D)))
    table_rep = jnp.tile(table_pad, (_REP, 1))
    cc = c.astype(jnp.int32)
    c_adj = (cc + _C * (jnp.arange(_B, dtype=jnp.int32) % _REP)).reshape(
        _B // _CHUNK, _CHUNK)
    mask_rows = _sc_mask_gather()(table_rep, c_adj)
    emb_t, embnorm = _tc_matmul(x, W.T, b.reshape(1, _D))
    masked_t, masknorm = _tc_epilogue(emb_t, mask_rows)
    return (masked_t.T, masknorm.reshape(()), embnorm.reshape(()), emb_t.T)
